# baseline (speedup 1.0000x reference)
import functools

import jax
import jax.numpy as jnp
from jax import lax
from jax.experimental import pallas as pl
from jax.experimental.pallas import tpu as pltpu
from jax.experimental.pallas import tpu_sc as plsc

N_SC_CORES = 2
N_SUBCORES = 16
CHUNK = 128
ROW = 144
FEATS = 128
BM = 512


def _sc_aggregate(feat_aug, src4, dst4, zeros, n_pad, k_chunks):
    mesh = plsc.VectorSubcoreMesh(core_axis_name="c", subcore_axis_name="s")

    @functools.partial(
        pl.kernel,
        out_type=jax.ShapeDtypeStruct((N_SC_CORES, n_pad, ROW), jnp.float32),
        mesh=mesh,
        scratch_types=[
            pltpu.VMEM((k_chunks, CHUNK), jnp.int32),
            pltpu.VMEM((k_chunks, CHUNK), jnp.int32),
            pltpu.VMEM((CHUNK, ROW), jnp.float32),
            pltpu.VMEM_SHARED((n_pad, ROW), jnp.float32),
            pltpu.SemaphoreType.DMA,
        ],
        compiler_params=pltpu.CompilerParams(use_tc_tiling_on_sc=False),
    )
    def agg(feat_hbm, src_hbm, dst_hbm, zeros_hbm, out_hbm,
            src_v, dst_v, rows_v, acc_sh, sem):
        cid = lax.axis_index("c")
        sid = lax.axis_index("s")
        slab = n_pad // N_SUBCORES

        pltpu.sync_copy(zeros_hbm.at[pl.ds(sid * slab, slab)],
                        acc_sh.at[pl.ds(sid * slab, slab)])
        pltpu.sync_copy(src_hbm.at[cid, sid], src_v)
        pltpu.sync_copy(dst_hbm.at[cid, sid], dst_v)
        plsc.subcore_barrier()

        def body(j, carry):
            pltpu.async_copy(feat_hbm.at[src_v.at[j]], rows_v, sem).wait()
            pltpu.sync_copy(rows_v, acc_sh.at[dst_v.at[j]], add=True)
            return carry

        lax.fori_loop(0, k_chunks, body, 0)
        plsc.subcore_barrier()

        pltpu.sync_copy(acc_sh.at[pl.ds(sid * slab, slab)],
                        out_hbm.at[cid, pl.ds(sid * slab, slab)])

    return agg(feat_aug, src4, dst4, zeros)


def _tc_project(acc, onehot, gate_w, gate_b_pad, n_pad):
    grid = n_pad // BM

    def body(acc_ref, oh_ref, w_ref, b_ref, out_ref):
        a = acc_ref[0] + acc_ref[1]
        inv = 1.0 / jnp.maximum(a[:, FEATS:FEATS + 1], 1.0)
        neigh = a[:, :FEATS] * inv
        oh = oh_ref[...]
        p = jnp.dot(oh, b_ref[...], preferred_element_type=jnp.float32)
        for k in range(gate_w.shape[0]):
            xk = oh[:, k:k + 1] * neigh
            p = p + jnp.dot(xk, w_ref[k], preferred_element_type=jnp.float32)
        out_ref[...] = p

    return pl.pallas_call(
        body,
        grid=(grid,),
        in_specs=[
            pl.BlockSpec((N_SC_CORES, BM, ROW), lambda i: (0, i, 0)),
            pl.BlockSpec((BM, 16), lambda i: (i, 0)),
            pl.BlockSpec(gate_w.shape, lambda i: (0, 0, 0)),
            pl.BlockSpec(gate_b_pad.shape, lambda i: (0, 0)),
        ],
        out_specs=pl.BlockSpec((BM, gate_w.shape[2]), lambda i: (i, 0)),
        out_shape=jax.ShapeDtypeStruct((n_pad, gate_w.shape[2]), jnp.float32),
    )(acc, onehot, gate_w, gate_b_pad)


def kernel(feat, edge_index, ntype2, gate_W, gate_b, act_flag):
    n, f = feat.shape
    e = edge_index.shape[1]
    in_dim = gate_W.shape[0]

    n_pad = ((n + 16) + BM - 1) // BM * BM

    per_round = N_SC_CORES * N_SUBCORES * CHUNK
    k_chunks = (e + per_round - 1) // per_round
    e_pad = k_chunks * per_round
    src = jnp.concatenate(
        [edge_index[0], jnp.zeros((e_pad - e,), jnp.int32)])
    dst = jnp.concatenate(
        [edge_index[1], jnp.full((e_pad - e,), n, jnp.int32)])
    src4 = src.reshape(N_SC_CORES, N_SUBCORES, k_chunks, CHUNK)
    dst4 = dst.reshape(N_SC_CORES, N_SUBCORES, k_chunks, CHUNK)

    ones = jnp.ones((n, 1), jnp.float32)
    zpad = jnp.zeros((n, ROW - FEATS - 1), jnp.float32)
    feat_aug = jnp.concatenate([feat, ones, zpad], axis=1)

    zeros = jnp.zeros((n_pad, ROW), jnp.float32)
    acc = _sc_aggregate(feat_aug, src4, dst4, zeros, n_pad, k_chunks)

    oh = (ntype2[:, None] == jnp.arange(16, dtype=jnp.int32)[None, :]
          ).astype(jnp.float32)
    oh = jnp.pad(oh, ((0, n_pad - n), (0, 0)))
    gate_b_pad = jnp.zeros((16, gate_b.shape[1]), jnp.float32).at[:in_dim].set(gate_b)

    rst = _tc_project(acc, oh, gate_W, gate_b_pad, n_pad)
    return rst[:n]


# speedup vs baseline: 1.0277x; 1.0277x over previous
import functools

import jax
import jax.numpy as jnp
from jax import lax
from jax.experimental import pallas as pl
from jax.experimental.pallas import tpu as pltpu
from jax.experimental.pallas import tpu_sc as plsc

N_SC_CORES = 2
N_SUBCORES = 16
CHUNK = 128
ROW = 80
HALF = 64
NBUF = 3
BM = 512


def _sc_aggregate(feat_ab, srcx, dst3, zeros, n_pad, k_chunks):
    mesh = plsc.VectorSubcoreMesh(core_axis_name="c", subcore_axis_name="s")

    @functools.partial(
        pl.kernel,
        out_type=jax.ShapeDtypeStruct((N_SC_CORES, n_pad, ROW), jnp.float32),
        mesh=mesh,
        scratch_types=[
            pltpu.VMEM((k_chunks, CHUNK), jnp.int32),
            pltpu.VMEM((k_chunks, CHUNK), jnp.int32),
            pltpu.VMEM((CHUNK, ROW), jnp.float32),
            pltpu.VMEM((CHUNK, ROW), jnp.float32),
            pltpu.VMEM((CHUNK, ROW), jnp.float32),
            pltpu.VMEM((CHUNK, ROW), jnp.float32),
            pltpu.VMEM_SHARED((n_pad, ROW), jnp.float32),
            pltpu.SemaphoreType.DMA,
            pltpu.SemaphoreType.DMA,
            pltpu.SemaphoreType.DMA,
            pltpu.SemaphoreType.DMA,
        ],
        compiler_params=pltpu.CompilerParams(use_tc_tiling_on_sc=False),
    )
    def agg(feat_hbm, srcx_hbm, dst_hbm, zeros_hbm, out_hbm,
            src_v, dst_v, rows0_v, rows1_v, rows2_v, rows3_v, acc_sh,
            sem0, sem1, sem2, sem3):
        cid = lax.axis_index("c")
        sid = lax.axis_index("s")
        slab = n_pad // N_SUBCORES

        pltpu.sync_copy(zeros_hbm.at[pl.ds(sid * slab, slab)],
                        acc_sh.at[pl.ds(sid * slab, slab)])
        pltpu.sync_copy(srcx_hbm.at[cid, sid], src_v)
        pltpu.sync_copy(dst_hbm.at[sid], dst_v)
        plsc.subcore_barrier()

        bufs = (rows0_v, rows1_v, rows2_v, rows3_v)
        sems = (sem0, sem1, sem2, sem3)

        def gather_start(j, b):
            pltpu.async_copy(feat_hbm.at[src_v.at[j]], bufs[b], sems[b])

        def gather_wait(b):
            pltpu.make_async_copy(feat_hbm.at[src_v.at[0]], bufs[b],
                                  sems[b]).wait()

        for b in range(NBUF):
            gather_start(b, b)

        def body(i, carry):
            j = NBUF * i
            for b in range(NBUF):
                gather_wait(b)
                pltpu.sync_copy(bufs[b], acc_sh.at[dst_v.at[j + b]],
                                add=True)

                @pl.when(j + b + NBUF < k_chunks)
                def _():
                    gather_start(j + b + NBUF, b)
            return carry

        lax.fori_loop(0, k_chunks // NBUF, body, 0)
        plsc.subcore_barrier()

        pltpu.sync_copy(acc_sh.at[pl.ds(sid * slab, slab)],
                        out_hbm.at[cid, pl.ds(sid * slab, slab)])

    return agg(feat_ab, srcx, dst3, zeros)


def _tc_project(acc, onehot, gate_w, gate_b_pad, n_pad):
    grid = n_pad // BM

    def body(acc_ref, oh_ref, w_ref, b_ref, out_ref):
        a0 = acc_ref[0]
        a1 = acc_ref[1]
        inv = 1.0 / jnp.maximum(a0[:, HALF:HALF + 1], 1.0)
        neigh = jnp.concatenate([a0[:, :HALF], a1[:, :HALF]], axis=1) * inv
        oh = oh_ref[...]
        p = jnp.dot(oh, b_ref[...], preferred_element_type=jnp.float32)
        for k in range(gate_w.shape[0]):
            xk = oh[:, k:k + 1] * neigh
            p = p + jnp.dot(xk, w_ref[k], preferred_element_type=jnp.float32)
        out_ref[...] = p

    return pl.pallas_call(
        body,
        grid=(grid,),
        in_specs=[
            pl.BlockSpec((N_SC_CORES, BM, ROW), lambda i: (0, i, 0)),
            pl.BlockSpec((BM, 16), lambda i: (i, 0)),
            pl.BlockSpec(gate_w.shape, lambda i: (0, 0, 0)),
            pl.BlockSpec(gate_b_pad.shape, lambda i: (0, 0)),
        ],
        out_specs=pl.BlockSpec((BM, gate_w.shape[2]), lambda i: (i, 0)),
        out_shape=jax.ShapeDtypeStruct((n_pad, gate_w.shape[2]), jnp.float32),
    )(acc, onehot, gate_w, gate_b_pad)


def kernel(feat, edge_index, ntype2, gate_W, gate_b, act_flag):
    n, f = feat.shape
    e = edge_index.shape[1]
    in_dim = gate_W.shape[0]

    n_pad = ((n + 16) + BM - 1) // BM * BM

    per_round = N_SUBCORES * CHUNK
    k_chunks = (e + per_round - 1) // per_round
    k_chunks = (k_chunks + NBUF - 1) // NBUF * NBUF
    e_pad = k_chunks * per_round
    src = jnp.concatenate(
        [edge_index[0], jnp.zeros((e_pad - e,), jnp.int32)])
    dst = jnp.concatenate(
        [edge_index[1], jnp.full((e_pad - e,), n, jnp.int32)])
    src3 = src.reshape(N_SUBCORES, k_chunks, CHUNK)
    srcx = jnp.stack([src3, src3 + n])
    dst3 = dst.reshape(N_SUBCORES, k_chunks, CHUNK)

    ones = jnp.ones((n, 1), jnp.float32)
    zpad = jnp.zeros((n, ROW - HALF - 1), jnp.float32)
    feat_ab = jnp.concatenate([
        jnp.concatenate([feat[:, :HALF], ones, zpad], axis=1),
        jnp.concatenate([feat[:, HALF:], ones, zpad], axis=1),
    ], axis=0)

    zeros = jnp.zeros((n_pad, ROW), jnp.float32)
    acc = _sc_aggregate(feat_ab, srcx, dst3, zeros, n_pad, k_chunks)

    oh = (ntype2[:, None] == jnp.arange(16, dtype=jnp.int32)[None, :]
          ).astype(jnp.float32)
    oh = jnp.pad(oh, ((0, n_pad - n), (0, 0)))
    gate_b_pad = jnp.zeros((16, gate_b.shape[1]), jnp.float32).at[:in_dim].set(gate_b)

    rst = _tc_project(acc, oh, gate_W, gate_b_pad, n_pad)
    return rst[:n]


# speedup vs baseline: 1.1770x; 1.1453x over previous
import functools

import jax
import jax.numpy as jnp
from jax import lax
from jax.experimental import pallas as pl
from jax.experimental.pallas import tpu as pltpu
from jax.experimental.pallas import tpu_sc as plsc

N_SC_CORES = 2
N_SUBCORES = 16
CHUNK = 128
FEATS = 128
BM = 512


def _sc_aggregate(feat, src4, dst4, zeros, zeros1, n_pad, k_chunks):
    mesh = plsc.VectorSubcoreMesh(core_axis_name="c", subcore_axis_name="s")

    @functools.partial(
        pl.kernel,
        out_type=(
            jax.ShapeDtypeStruct((N_SC_CORES, n_pad, FEATS), jnp.float32),
            jax.ShapeDtypeStruct((N_SC_CORES, n_pad), jnp.float32),
        ),
        mesh=mesh,
        scratch_types=[
            pltpu.VMEM((k_chunks, CHUNK), jnp.int32),
            pltpu.VMEM((k_chunks, CHUNK), jnp.int32),
            pltpu.VMEM((CHUNK, FEATS), jnp.float32),
            pltpu.VMEM((CHUNK,), jnp.float32),
            pltpu.VMEM_SHARED((n_pad, FEATS), jnp.float32),
            pltpu.VMEM_SHARED((n_pad,), jnp.float32),
            pltpu.SemaphoreType.DMA,
        ],
        compiler_params=pltpu.CompilerParams(use_tc_tiling_on_sc=False),
    )
    def agg(feat_hbm, src_hbm, dst_hbm, zeros_hbm, zeros1_hbm,
            sums_hbm, degs_hbm,
            src_v, dst_v, rows_v, ones_v, acc_sh, deg_sh, sem):
        cid = lax.axis_index("c")
        sid = lax.axis_index("s")
        slab = n_pad // N_SUBCORES

        def fill(i, carry):
            ones_v[pl.ds(i * 16, 16)] = jnp.ones((16,), jnp.float32)
            return carry

        lax.fori_loop(0, CHUNK // 16, fill, 0)

        pltpu.sync_copy(zeros_hbm.at[pl.ds(sid * slab, slab)],
                        acc_sh.at[pl.ds(sid * slab, slab)])
        pltpu.sync_copy(zeros1_hbm.at[pl.ds(sid * slab, slab)],
                        deg_sh.at[pl.ds(sid * slab, slab)])
        pltpu.sync_copy(src_hbm.at[cid, sid], src_v)
        pltpu.sync_copy(dst_hbm.at[cid, sid], dst_v)
        plsc.subcore_barrier()

        def body(j, carry):
            pltpu.async_copy(feat_hbm.at[src_v.at[j]], rows_v, sem).wait()
            pltpu.sync_copy(ones_v, deg_sh.at[dst_v.at[j]], add=True)
            pltpu.sync_copy(rows_v, acc_sh.at[dst_v.at[j]], add=True)
            return carry

        lax.fori_loop(0, k_chunks, body, 0)
        plsc.subcore_barrier()

        pltpu.sync_copy(acc_sh.at[pl.ds(sid * slab, slab)],
                        sums_hbm.at[cid, pl.ds(sid * slab, slab)])
        pltpu.sync_copy(deg_sh.at[pl.ds(sid * slab, slab)],
                        degs_hbm.at[cid, pl.ds(sid * slab, slab)])

    return agg(feat, src4, dst4, zeros, zeros1)


def _tc_project(sums, degs, onehot, gate_w, gate_b_pad, n_pad):
    grid = n_pad // BM

    def body(sums_ref, degs_ref, oh_ref, w_ref, b_ref, out_ref):
        a = sums_ref[0] + sums_ref[1]
        d = degs_ref[0] + degs_ref[1]
        neigh = a * (1.0 / jnp.maximum(d, 1.0))
        oh = oh_ref[...]
        p = jnp.dot(oh, b_ref[...], preferred_element_type=jnp.float32)
        for k in range(gate_w.shape[0]):
            xk = oh[:, k:k + 1] * neigh
            p = p + jnp.dot(xk, w_ref[k], preferred_element_type=jnp.float32)
        out_ref[...] = p

    return pl.pallas_call(
        body,
        grid=(grid,),
        in_specs=[
            pl.BlockSpec((N_SC_CORES, BM, FEATS), lambda i: (0, i, 0)),
            pl.BlockSpec((N_SC_CORES, BM, 1), lambda i: (0, i, 0)),
            pl.BlockSpec((BM, 16), lambda i: (i, 0)),
            pl.BlockSpec(gate_w.shape, lambda i: (0, 0, 0)),
            pl.BlockSpec(gate_b_pad.shape, lambda i: (0, 0)),
        ],
        out_specs=pl.BlockSpec((BM, gate_w.shape[2]), lambda i: (i, 0)),
        out_shape=jax.ShapeDtypeStruct((n_pad, gate_w.shape[2]), jnp.float32),
    )(sums, degs, onehot, gate_w, gate_b_pad)


def kernel(feat, edge_index, ntype2, gate_W, gate_b, act_flag):
    n, f = feat.shape
    e = edge_index.shape[1]
    in_dim = gate_W.shape[0]

    n_pad = ((n + 16) + BM - 1) // BM * BM

    per_round = N_SC_CORES * N_SUBCORES * CHUNK
    k_chunks = (e + per_round - 1) // per_round
    e_pad = k_chunks * per_round
    src = jnp.concatenate(
        [edge_index[0], jnp.zeros((e_pad - e,), jnp.int32)])
    dst = jnp.concatenate(
        [edge_index[1], jnp.full((e_pad - e,), n, jnp.int32)])
    src4 = src.reshape(N_SC_CORES, N_SUBCORES, k_chunks, CHUNK)
    dst4 = dst.reshape(N_SC_CORES, N_SUBCORES, k_chunks, CHUNK)

    zeros = jnp.zeros((n_pad, FEATS), jnp.float32)
    zeros1 = jnp.zeros((n_pad,), jnp.float32)
    sums, degs = _sc_aggregate(feat, src4, dst4, zeros, zeros1,
                               n_pad, k_chunks)
    degs = degs.reshape(N_SC_CORES, n_pad, 1)

    oh = (ntype2[:, None] == jnp.arange(16, dtype=jnp.int32)[None, :]
          ).astype(jnp.float32)
    oh = jnp.pad(oh, ((0, n_pad - n), (0, 0)))
    gate_b_pad = jnp.zeros((16, gate_b.shape[1]), jnp.float32).at[:in_dim].set(gate_b)

    rst = _tc_project(sums, degs, oh, gate_W, gate_b_pad, n_pad)
    return rst[:n]


# speedup vs baseline: 1.3998x; 1.1892x over previous
import functools

import jax
import jax.numpy as jnp
from jax import lax
from jax.experimental import pallas as pl
from jax.experimental.pallas import tpu as pltpu
from jax.experimental.pallas import tpu_sc as plsc

N_SC_CORES = 2
N_SUBCORES = 16
CHUNK = 128
HALF = 64
BM = 512


def _sc_aggregate(feat_ab, src4, dst4, zeros, zeros1, n, n_pad, k_chunks):
    mesh = plsc.VectorSubcoreMesh(core_axis_name="c", subcore_axis_name="s")

    @functools.partial(
        pl.kernel,
        out_type=(
            jax.ShapeDtypeStruct((N_SC_CORES, n_pad, HALF), jnp.float32),
            jax.ShapeDtypeStruct((N_SC_CORES, n_pad), jnp.float32),
        ),
        mesh=mesh,
        scratch_types=[
            pltpu.VMEM((k_chunks, CHUNK), jnp.int32),
            pltpu.VMEM((k_chunks, CHUNK), jnp.int32),
            pltpu.VMEM((CHUNK, HALF), jnp.float32),
            pltpu.VMEM((CHUNK,), jnp.float32),
            pltpu.VMEM_SHARED((n, HALF), jnp.float32),
            pltpu.VMEM_SHARED((n_pad, HALF), jnp.float32),
            pltpu.VMEM_SHARED((n_pad,), jnp.float32),
            pltpu.SemaphoreType.DMA,
        ],
        compiler_params=pltpu.CompilerParams(use_tc_tiling_on_sc=False),
    )
    def agg(feat_hbm, src_hbm, dst_hbm, zeros_hbm, zeros1_hbm,
            sums_hbm, degs_hbm,
            src_v, dst_v, rows_v, ones_v, feat_sh, acc_sh, deg_sh, sem):
        cid = lax.axis_index("c")
        sid = lax.axis_index("s")
        slab = n_pad // N_SUBCORES
        fslab = n // N_SUBCORES

        def fill(i, carry):
            ones_v[pl.ds(i * 16, 16)] = jnp.ones((16,), jnp.float32)
            return carry

        lax.fori_loop(0, CHUNK // 16, fill, 0)

        pltpu.sync_copy(feat_hbm.at[cid, pl.ds(sid * fslab, fslab)],
                        feat_sh.at[pl.ds(sid * fslab, fslab)])
        pltpu.sync_copy(zeros_hbm.at[pl.ds(sid * slab, slab)],
                        acc_sh.at[pl.ds(sid * slab, slab)])
        pltpu.sync_copy(zeros1_hbm.at[pl.ds(sid * slab, slab)],
                        deg_sh.at[pl.ds(sid * slab, slab)])
        pltpu.sync_copy(src_hbm.at[sid], src_v)
        pltpu.sync_copy(dst_hbm.at[sid], dst_v)
        plsc.subcore_barrier()

        def body(j, carry):
            pltpu.async_copy(feat_sh.at[src_v.at[j]], rows_v, sem).wait()
            pltpu.sync_copy(ones_v, deg_sh.at[dst_v.at[j]], add=True)
            pltpu.sync_copy(rows_v, acc_sh.at[dst_v.at[j]], add=True)
            return carry

        lax.fori_loop(0, k_chunks, body, 0)
        plsc.subcore_barrier()

        pltpu.sync_copy(acc_sh.at[pl.ds(sid * slab, slab)],
                        sums_hbm.at[cid, pl.ds(sid * slab, slab)])
        pltpu.sync_copy(deg_sh.at[pl.ds(sid * slab, slab)],
                        degs_hbm.at[cid, pl.ds(sid * slab, slab)])

    return agg(feat_ab, src4, dst4, zeros, zeros1)


def _tc_project(sums, degs, onehot, gate_w, gate_b_pad, n_pad):
    grid = n_pad // BM

    def body(sums_ref, degs_ref, oh_ref, w_ref, b_ref, out_ref):
        a0 = sums_ref[0]
        a1 = sums_ref[1]
        d = degs_ref[0]
        inv = 1.0 / jnp.maximum(d, 1.0)
        neigh = jnp.concatenate([a0, a1], axis=1) * inv
        oh = oh_ref[...]
        p = jnp.dot(oh, b_ref[...], preferred_element_type=jnp.float32)
        for k in range(gate_w.shape[0]):
            xk = oh[:, k:k + 1] * neigh
            p = p + jnp.dot(xk, w_ref[k], preferred_element_type=jnp.float32)
        out_ref[...] = p

    return pl.pallas_call(
        body,
        grid=(grid,),
        in_specs=[
            pl.BlockSpec((N_SC_CORES, BM, HALF), lambda i: (0, i, 0)),
            pl.BlockSpec((1, BM, 1), lambda i: (0, i, 0)),
            pl.BlockSpec((BM, 16), lambda i: (i, 0)),
            pl.BlockSpec(gate_w.shape, lambda i: (0, 0, 0)),
            pl.BlockSpec(gate_b_pad.shape, lambda i: (0, 0)),
        ],
        out_specs=pl.BlockSpec((BM, gate_w.shape[2]), lambda i: (i, 0)),
        out_shape=jax.ShapeDtypeStruct((n_pad, gate_w.shape[2]), jnp.float32),
    )(sums, degs, onehot, gate_w, gate_b_pad)


def kernel(feat, edge_index, ntype2, gate_W, gate_b, act_flag):
    n, f = feat.shape
    e = edge_index.shape[1]
    in_dim = gate_W.shape[0]

    n_pad = ((n + 16) + BM - 1) // BM * BM

    per_round = N_SUBCORES * CHUNK
    k_chunks = (e + per_round - 1) // per_round
    e_pad = k_chunks * per_round
    src = jnp.concatenate(
        [edge_index[0], jnp.zeros((e_pad - e,), jnp.int32)])
    dst = jnp.concatenate(
        [edge_index[1], jnp.full((e_pad - e,), n, jnp.int32)])
    src4 = src.reshape(N_SUBCORES, k_chunks, CHUNK)
    dst4 = dst.reshape(N_SUBCORES, k_chunks, CHUNK)

    feat_ab = jnp.stack([feat[:, :HALF], feat[:, HALF:]])

    zeros = jnp.zeros((n_pad, HALF), jnp.float32)
    zeros1 = jnp.zeros((n_pad,), jnp.float32)
    sums, degs = _sc_aggregate(feat_ab, src4, dst4, zeros, zeros1,
                               n, n_pad, k_chunks)
    degs = degs.reshape(N_SC_CORES, n_pad, 1)

    oh = (ntype2[:, None] == jnp.arange(16, dtype=jnp.int32)[None, :]
          ).astype(jnp.float32)
    oh = jnp.pad(oh, ((0, n_pad - n), (0, 0)))
    gate_b_pad = jnp.zeros((16, gate_b.shape[1]), jnp.float32).at[:in_dim].set(gate_b)

    rst = _tc_project(sums, degs, oh, gate_W, gate_b_pad, n_pad)
    return rst[:n]


# speedup vs baseline: 1.4479x; 1.0344x over previous
import functools

import jax
import jax.numpy as jnp
from jax import lax
from jax.experimental import pallas as pl
from jax.experimental.pallas import tpu as pltpu
from jax.experimental.pallas import tpu_sc as plsc

N_SC_CORES = 2
N_SUBCORES = 16
CHUNK = 128
HALF = 64
BM = 512


def _sc_aggregate(feat_ab, src4, dst4, zeros, zeros1, n, n_pad, k_chunks):
    mesh = plsc.VectorSubcoreMesh(core_axis_name="c", subcore_axis_name="s")

    @functools.partial(
        pl.kernel,
        out_type=(
            jax.ShapeDtypeStruct((N_SC_CORES, n_pad, HALF), jnp.float32),
            jax.ShapeDtypeStruct((N_SC_CORES, n_pad), jnp.float32),
        ),
        mesh=mesh,
        scratch_types=[
            pltpu.VMEM((k_chunks, CHUNK), jnp.int32),
            pltpu.VMEM((k_chunks, CHUNK), jnp.int32),
            pltpu.VMEM((CHUNK, HALF), jnp.float32),
            pltpu.VMEM((CHUNK,), jnp.float32),
            pltpu.VMEM_SHARED((n, HALF), jnp.float32),
            pltpu.VMEM_SHARED((n_pad, HALF), jnp.float32),
            pltpu.VMEM_SHARED((n_pad,), jnp.float32),
            pltpu.SemaphoreType.DMA,
        ],
        compiler_params=pltpu.CompilerParams(use_tc_tiling_on_sc=False),
    )
    def agg(feat_hbm, src_hbm, dst_hbm, zeros_hbm, zeros1_hbm,
            sums_hbm, degs_hbm,
            src_v, dst_v, rows_v, ones_v, feat_sh, acc_sh, deg_sh, sem):
        cid = lax.axis_index("c")
        sid = lax.axis_index("s")
        slab = n_pad // N_SUBCORES
        fslab = n // N_SUBCORES

        def fill(i, carry):
            ones_v[pl.ds(i * 16, 16)] = jnp.ones((16,), jnp.float32)
            return carry

        lax.fori_loop(0, CHUNK // 16, fill, 0)

        pltpu.sync_copy(feat_hbm.at[cid, pl.ds(sid * fslab, fslab)],
                        feat_sh.at[pl.ds(sid * fslab, fslab)])
        pltpu.sync_copy(zeros_hbm.at[pl.ds(sid * slab, slab)],
                        acc_sh.at[pl.ds(sid * slab, slab)])
        pltpu.sync_copy(zeros1_hbm.at[pl.ds(sid * slab, slab)],
                        deg_sh.at[pl.ds(sid * slab, slab)])
        pltpu.sync_copy(src_hbm.at[sid], src_v)
        pltpu.sync_copy(dst_hbm.at[sid], dst_v)
        plsc.subcore_barrier()

        k_half = k_chunks // 2

        def body(j, carry):
            pltpu.async_copy(feat_sh.at[src_v.at[j]], rows_v, sem).wait()

            @pl.when(jnp.where(cid == 0, j < k_half, j >= k_half))
            def _():
                pltpu.sync_copy(ones_v, deg_sh.at[dst_v.at[j]], add=True)

            pltpu.sync_copy(rows_v, acc_sh.at[dst_v.at[j]], add=True)
            return carry

        lax.fori_loop(0, k_chunks, body, 0)
        plsc.subcore_barrier()

        pltpu.sync_copy(acc_sh.at[pl.ds(sid * slab, slab)],
                        sums_hbm.at[cid, pl.ds(sid * slab, slab)])
        pltpu.sync_copy(deg_sh.at[pl.ds(sid * slab, slab)],
                        degs_hbm.at[cid, pl.ds(sid * slab, slab)])

    return agg(feat_ab, src4, dst4, zeros, zeros1)


def _tc_project(sums, degs, onehot, gate_w, gate_b_pad, n_pad, n):
    grid = n_pad // BM

    def body(sums_ref, degs_ref, oh_ref, w_ref, b_ref, out_ref):
        a0 = sums_ref[0]
        a1 = sums_ref[1]
        d = degs_ref[0] + degs_ref[1]
        inv = 1.0 / jnp.maximum(d, 1.0)
        neigh = jnp.concatenate([a0, a1], axis=1) * inv
        oh = oh_ref[...]
        p = jnp.dot(oh, b_ref[...], preferred_element_type=jnp.float32)
        for k in range(gate_w.shape[0]):
            xk = oh[:, k:k + 1] * neigh
            p = p + jnp.dot(xk, w_ref[k], preferred_element_type=jnp.float32)
        out_ref[...] = p

    return pl.pallas_call(
        body,
        grid=(grid,),
        in_specs=[
            pl.BlockSpec((N_SC_CORES, BM, HALF), lambda i: (0, i, 0)),
            pl.BlockSpec((N_SC_CORES, BM, 1), lambda i: (0, i, 0)),
            pl.BlockSpec((BM, 16), lambda i: (i, 0)),
            pl.BlockSpec(gate_w.shape, lambda i: (0, 0, 0)),
            pl.BlockSpec(gate_b_pad.shape, lambda i: (0, 0)),
        ],
        out_specs=pl.BlockSpec((BM, gate_w.shape[2]), lambda i: (i, 0)),
        out_shape=jax.ShapeDtypeStruct((n, gate_w.shape[2]), jnp.float32),
    )(sums, degs, onehot, gate_w, gate_b_pad)


def kernel(feat, edge_index, ntype2, gate_W, gate_b, act_flag):
    n, f = feat.shape
    e = edge_index.shape[1]
    in_dim = gate_W.shape[0]

    n_pad = ((n + 16) + BM - 1) // BM * BM

    per_round = N_SUBCORES * CHUNK
    k_chunks = (e + per_round - 1) // per_round
    e_pad = k_chunks * per_round
    src = jnp.concatenate(
        [edge_index[0], jnp.zeros((e_pad - e,), jnp.int32)])
    dst = jnp.concatenate(
        [edge_index[1], jnp.full((e_pad - e,), n, jnp.int32)])
    src4 = src.reshape(N_SUBCORES, k_chunks, CHUNK)
    dst4 = dst.reshape(N_SUBCORES, k_chunks, CHUNK)

    feat_ab = jnp.stack([feat[:, :HALF], feat[:, HALF:]])

    zeros = jnp.zeros((n_pad, HALF), jnp.float32)
    zeros1 = jnp.zeros((n_pad,), jnp.float32)
    sums, degs = _sc_aggregate(feat_ab, src4, dst4, zeros, zeros1,
                               n, n_pad, k_chunks)
    degs = degs.reshape(N_SC_CORES, n_pad, 1)

    oh = (ntype2[:, None] == jnp.arange(16, dtype=jnp.int32)[None, :]
          ).astype(jnp.float32)
    oh = jnp.pad(oh, ((0, n_pad - n), (0, 0)))
    gate_b_pad = jnp.zeros((16, gate_b.shape[1]), jnp.float32).at[:in_dim].set(gate_b)

    return _tc_project(sums, degs, oh, gate_W, gate_b_pad, n_pad, n)


# speedup vs baseline: 1.5276x; 1.0550x over previous
import functools

import jax
import jax.numpy as jnp
from jax import lax
from jax.experimental import pallas as pl
from jax.experimental.pallas import tpu as pltpu
from jax.experimental.pallas import tpu_sc as plsc

N_SC_CORES = 2
N_SUBCORES = 16
CHUNK = 128
HALF = 64
BM = 512


def _sc_aggregate(feat, src4, dst4, zeros, zeros1, n, n_pad, k_chunks):
    mesh = plsc.VectorSubcoreMesh(core_axis_name="c", subcore_axis_name="s")

    @functools.partial(
        pl.kernel,
        out_type=(
            jax.ShapeDtypeStruct((N_SC_CORES, n_pad, HALF), jnp.float32),
            jax.ShapeDtypeStruct((N_SC_CORES, n_pad), jnp.float32),
        ),
        mesh=mesh,
        scratch_types=[
            pltpu.VMEM((k_chunks, CHUNK), jnp.int32),
            pltpu.VMEM((k_chunks, CHUNK), jnp.int32),
            pltpu.VMEM((CHUNK, HALF), jnp.float32),
            pltpu.VMEM((CHUNK,), jnp.float32),
            pltpu.VMEM_SHARED((n, HALF), jnp.float32),
            pltpu.VMEM_SHARED((n_pad, HALF), jnp.float32),
            pltpu.VMEM_SHARED((n_pad,), jnp.float32),
            pltpu.SemaphoreType.DMA,
        ],
        compiler_params=pltpu.CompilerParams(use_tc_tiling_on_sc=False),
    )
    def agg(feat_hbm, src_hbm, dst_hbm, zeros_hbm, zeros1_hbm,
            sums_hbm, degs_hbm,
            src_v, dst_v, rows_v, ones_v, feat_sh, acc_sh, deg_sh, sem):
        cid = lax.axis_index("c")
        sid = lax.axis_index("s")
        slab = n_pad // N_SUBCORES
        fslab = n // N_SUBCORES

        def fill(i, carry):
            ones_v[pl.ds(i * 16, 16)] = jnp.ones((16,), jnp.float32)
            return carry

        lax.fori_loop(0, CHUNK // 16, fill, 0)

        pltpu.sync_copy(
            feat_hbm.at[pl.ds(sid * fslab, fslab), pl.ds(cid * HALF, HALF)],
            feat_sh.at[pl.ds(sid * fslab, fslab)])
        pltpu.sync_copy(zeros_hbm.at[pl.ds(sid * slab, slab)],
                        acc_sh.at[pl.ds(sid * slab, slab)])
        pltpu.sync_copy(zeros1_hbm.at[pl.ds(sid * slab, slab)],
                        deg_sh.at[pl.ds(sid * slab, slab)])
        pltpu.sync_copy(src_hbm.at[sid], src_v)
        pltpu.sync_copy(dst_hbm.at[sid], dst_v)
        plsc.subcore_barrier()

        k_half = k_chunks // 2

        def body(j, carry):
            pltpu.async_copy(feat_sh.at[src_v.at[j]], rows_v, sem).wait()

            @pl.when(jnp.where(cid == 0, j < k_half, j >= k_half))
            def _():
                pltpu.sync_copy(ones_v, deg_sh.at[dst_v.at[j]], add=True)

            pltpu.sync_copy(rows_v, acc_sh.at[dst_v.at[j]], add=True)
            return carry

        lax.fori_loop(0, k_chunks, body, 0)
        plsc.subcore_barrier()

        pltpu.sync_copy(acc_sh.at[pl.ds(sid * slab, slab)],
                        sums_hbm.at[cid, pl.ds(sid * slab, slab)])
        pltpu.sync_copy(deg_sh.at[pl.ds(sid * slab, slab)],
                        degs_hbm.at[cid, pl.ds(sid * slab, slab)])

    return agg(feat, src4, dst4, zeros, zeros1)


def _tc_project(sums, degs, onehot, gate_w, gate_b_pad, n_pad, n):
    grid = n_pad // BM

    def body(sums_ref, degs_ref, oh_ref, w_ref, b_ref, out_ref):
        a0 = sums_ref[0]
        a1 = sums_ref[1]
        d = degs_ref[0] + degs_ref[1]
        inv = 1.0 / jnp.maximum(d, 1.0)
        neigh = jnp.concatenate([a0, a1], axis=1) * inv
        oh = oh_ref[...]
        p = jnp.dot(oh, b_ref[...], preferred_element_type=jnp.float32)
        for k in range(gate_w.shape[0]):
            xk = oh[:, k:k + 1] * neigh
            p = p + jnp.dot(xk, w_ref[k], preferred_element_type=jnp.float32)
        out_ref[...] = p

    return pl.pallas_call(
        body,
        grid=(grid,),
        in_specs=[
            pl.BlockSpec((N_SC_CORES, BM, HALF), lambda i: (0, i, 0)),
            pl.BlockSpec((N_SC_CORES, BM, 1), lambda i: (0, i, 0)),
            pl.BlockSpec((BM, 16), lambda i: (i, 0)),
            pl.BlockSpec(gate_w.shape, lambda i: (0, 0, 0)),
            pl.BlockSpec(gate_b_pad.shape, lambda i: (0, 0)),
        ],
        out_specs=pl.BlockSpec((BM, gate_w.shape[2]), lambda i: (i, 0)),
        out_shape=jax.ShapeDtypeStruct((n, gate_w.shape[2]), jnp.float32),
    )(sums, degs, onehot, gate_w, gate_b_pad)


def kernel(feat, edge_index, ntype2, gate_W, gate_b, act_flag):
    n, f = feat.shape
    e = edge_index.shape[1]
    in_dim = gate_W.shape[0]

    n_pad = ((n + 16) + BM - 1) // BM * BM

    per_round = N_SUBCORES * CHUNK
    k_chunks = (e + per_round - 1) // per_round
    e_pad = k_chunks * per_round
    src = jnp.concatenate(
        [edge_index[0], jnp.zeros((e_pad - e,), jnp.int32)])
    dst = jnp.concatenate(
        [edge_index[1], jnp.full((e_pad - e,), n, jnp.int32)])
    src4 = src.reshape(N_SUBCORES, k_chunks, CHUNK)
    dst4 = dst.reshape(N_SUBCORES, k_chunks, CHUNK)

    zeros = jnp.zeros((n_pad, HALF), jnp.float32)
    zeros1 = jnp.zeros((n_pad,), jnp.float32)
    sums, degs = _sc_aggregate(feat, src4, dst4, zeros, zeros1,
                               n, n_pad, k_chunks)
    degs = degs.reshape(N_SC_CORES, n_pad, 1)

    oh = (ntype2[:, None] == jnp.arange(16, dtype=jnp.int32)[None, :]
          ).astype(jnp.float32)
    oh = jnp.pad(oh, ((0, n_pad - n), (0, 0)))
    gate_b_pad = jnp.zeros((16, gate_b.shape[1]), jnp.float32).at[:in_dim].set(gate_b)

    return _tc_project(sums, degs, oh, gate_W, gate_b_pad, n_pad, n)
